# baseline (speedup 1.0000x reference)
import jax
import jax.numpy as jnp
from jax import lax
from jax.experimental import pallas as pl
from jax.experimental.pallas import tpu as pltpu
from jax.experimental.pallas import tpu_sc as plsc

NU, NI = 40000, 10000
NN = NU + NI
EE = 800000
D = 64
HALF = NN // 2
NS = 16
NC = 2
ROWS_T = 1568
HPAD = NS * ROWS_T
NPAD = NC * HPAD
PAD88 = HPAD - HALF
DUMMY = HALF
CHUNK = 128
EDGES_T = EE // NS
NCHUNK = 392
ETILE = NCHUNK * CHUNK
EP = NS * ETILE
EPW = EP // (NC * NS)
PB = 6272
CW = NPAD // NS

_MESH = plsc.VectorSubcoreMesh(
    core_axis_name="c", subcore_axis_name="s", num_cores=NC, num_subcores=NS
)
_SC_PARAMS = pltpu.CompilerParams(
    needs_layout_passes=False, use_tc_tiling_on_sc=False
)


def _prep_body(rowp, colp, grow_o, lcol_o, degp_o, histp_o,
               rowb, colb, gb, l0b, l1b, histo, tmp, acc):
    c = lax.axis_index("c")
    s = lax.axis_index("s")
    w = c * NS + s
    base = w * EPW

    def zh(i, _):
        histo[pl.ds(i * 16, 16)] = jnp.zeros((16,), jnp.float32)
        return 0
    lax.fori_loop(0, NPAD // 16, zh, 0)

    for blk in range(EPW // PB):
        off = base + blk * PB
        pltpu.sync_copy(rowp.at[pl.ds(off, PB)], rowb)
        pltpu.sync_copy(colp.at[pl.ds(off, PB)], colb)

        def body(i, _):
            r16 = rowb[pl.ds(i * 16, 16)]
            c16 = colb[pl.ds(i * 16, 16)]
            gb[pl.ds(i * 16, 16)] = jnp.where(r16 >= HALF, r16 + PAD88, r16)
            l0b[pl.ds(i * 16, 16)] = jnp.where(c16 < HALF, c16, DUMMY)
            in1 = (c16 >= HALF) & (c16 < NN)
            l1b[pl.ds(i * 16, 16)] = jnp.where(in1, c16 - HALF, DUMMY)
            gcol = jnp.where(c16 < HALF, c16, c16 + PAD88)
            plsc.addupdate_scatter(histo, [gcol], jnp.ones((16,), jnp.float32))
            return 0
        lax.fori_loop(0, PB // 16, body, 0)

        pltpu.sync_copy(gb, grow_o.at[pl.ds(off, PB)])
        pltpu.sync_copy(l0b, lcol_o.at[0, pl.ds(off, PB)])
        pltpu.sync_copy(l1b, lcol_o.at[1, pl.ds(off, PB)])

    pltpu.sync_copy(histo, histp_o.at[c, s])
    plsc.subcore_barrier()

    def za(i, _):
        acc[pl.ds(i * 16, 16)] = jnp.zeros((16,), jnp.float32)
        return 0
    lax.fori_loop(0, CW // 16, za, 0)
    for j in range(NS):
        pltpu.sync_copy(histp_o.at[c, j, pl.ds(s * CW, CW)], tmp)

        def ab(i, _):
            acc[pl.ds(i * 16, 16)] = acc[pl.ds(i * 16, 16)] + tmp[pl.ds(i * 16, 16)]
            return 0
        lax.fori_loop(0, CW // 16, ab, 0)
    pltpu.sync_copy(acc, degp_o.at[c, s])


_prep_call = pl.kernel(
    _prep_body,
    out_type=(
        jax.ShapeDtypeStruct((EP,), jnp.int32),
        jax.ShapeDtypeStruct((2, EP), jnp.int32),
        jax.ShapeDtypeStruct((NC, NS, CW), jnp.float32),
        jax.ShapeDtypeStruct((NC, NS, NPAD), jnp.float32),
    ),
    mesh=_MESH,
    scratch_types=[
        pltpu.VMEM((PB,), jnp.int32),
        pltpu.VMEM((PB,), jnp.int32),
        pltpu.VMEM((PB,), jnp.int32),
        pltpu.VMEM((PB,), jnp.int32),
        pltpu.VMEM((PB,), jnp.int32),
        pltpu.VMEM((NPAD,), jnp.float32),
        pltpu.VMEM((CW,), jnp.float32),
        pltpu.VMEM((CW,), jnp.float32),
    ],
    compiler_params=_SC_PARAMS,
)


IBLK = 8
NBLK = NCHUNK // IBLK


def _layer_body(y_in, grow2, lcol3, z_out,
                gidx, cidx, buf0, buf1, accum, gsem_a, gsem_b):
    c = lax.axis_index("c")
    s = lax.axis_index("s")
    bufs = (buf0, buf1)
    gsems = (gsem_a, gsem_b)

    def zb(i, _):
        for q in range(4):
            buf0[i, pl.ds(q * 16, 16)] = jnp.zeros((16,), jnp.float32)
        return 0
    lax.fori_loop(0, CHUNK, zb, 0)
    r0 = s * ROWS_T
    for k in range(12):
        pltpu.sync_copy(buf0, accum.at[pl.ds(r0 + k * 128, 128)])
    pltpu.sync_copy(buf0.at[pl.ds(0, 32)], accum.at[pl.ds(r0 + 1536, 32)])
    plsc.subcore_barrier()

    def loop(ib, _):
        base = s * NCHUNK + ib * IBLK
        pltpu.sync_copy(grow2.at[pl.ds(base, IBLK)], gidx)
        pltpu.sync_copy(lcol3.at[c, pl.ds(base, IBLK)], cidx)
        pltpu.make_async_copy(y_in.at[gidx.at[0]], bufs[0], gsems[0]).start()
        for jj in range(IBLK):
            b = jj % 2
            if jj + 1 < IBLK:
                pltpu.make_async_copy(
                    y_in.at[gidx.at[jj + 1]], bufs[1 - b], gsems[1 - b]
                ).start()
            pltpu.make_async_copy(
                y_in.at[gidx.at[jj]], bufs[b], gsems[b]).wait()
            pltpu.sync_copy(bufs[b], accum.at[cidx.at[jj]], add=True)
        return 0
    lax.fori_loop(0, NBLK, loop, 0)
    plsc.subcore_barrier()

    zoff = c * HPAD + r0
    for k in range(12):
        pltpu.sync_copy(accum.at[pl.ds(r0 + k * 128, 128)], buf0)
        pltpu.sync_copy(buf0, z_out.at[pl.ds(zoff + k * 128, 128)])
    pltpu.sync_copy(accum.at[pl.ds(r0 + 1536, 32)], buf1.at[pl.ds(0, 32)])
    pltpu.sync_copy(buf1.at[pl.ds(0, 32)], z_out.at[pl.ds(zoff + 1536, 32)])


_layer_call = pl.kernel(
    _layer_body,
    out_type=jax.ShapeDtypeStruct((NPAD, D), jnp.float32),
    mesh=_MESH,
    scratch_types=[
        pltpu.VMEM((IBLK, CHUNK), jnp.int32),
        pltpu.VMEM((IBLK, CHUNK), jnp.int32),
        pltpu.VMEM((CHUNK, D), jnp.float32),
        pltpu.VMEM((CHUNK, D), jnp.float32),
        pltpu.VMEM_SHARED((HPAD, D), jnp.float32),
        pltpu.SemaphoreType.DMA,
        pltpu.SemaphoreType.DMA,
    ],
    compiler_params=_SC_PARAMS,
)


def _tc_prep_body(deg0_ref, deg1_ref, x0_ref, y0_ref, d_ref, d2_ref):
    deg = deg0_ref[...] + deg1_ref[...]
    dinv = jnp.where(deg > 0, 1.0 / jnp.sqrt(jnp.maximum(deg, 1.0)), 0.0)
    d_ref[...] = dinv
    d2_ref[...] = dinv * dinv
    y0_ref[...] = x0_ref[...] * dinv


def _tc_prep(deg0, deg1, x0):
    nb = NPAD // 512
    return pl.pallas_call(
        _tc_prep_body,
        grid=(nb,),
        in_specs=[
            pl.BlockSpec((512, 1), lambda i: (i, 0)),
            pl.BlockSpec((512, 1), lambda i: (i, 0)),
            pl.BlockSpec((512, D), lambda i: (i, 0)),
        ],
        out_specs=[
            pl.BlockSpec((512, D), lambda i: (i, 0)),
            pl.BlockSpec((512, 1), lambda i: (i, 0)),
            pl.BlockSpec((512, 1), lambda i: (i, 0)),
        ],
        out_shape=[
            jax.ShapeDtypeStruct((NPAD, D), jnp.float32),
            jax.ShapeDtypeStruct((NPAD, 1), jnp.float32),
            jax.ShapeDtypeStruct((NPAD, 1), jnp.float32),
        ],
    )(deg0, deg1, x0)


def _tc_scale_body(z_ref, d2_ref, y_ref):
    y_ref[...] = z_ref[...] * d2_ref[...]


def _tc_scale(z, d2):
    nb = NPAD // 512
    return pl.pallas_call(
        _tc_scale_body,
        grid=(nb,),
        in_specs=[
            pl.BlockSpec((512, D), lambda i: (i, 0)),
            pl.BlockSpec((512, 1), lambda i: (i, 0)),
        ],
        out_specs=pl.BlockSpec((512, D), lambda i: (i, 0)),
        out_shape=jax.ShapeDtypeStruct((NPAD, D), jnp.float32),
    )(z, d2)


def _tc_base_body(x0_ref, d_ref, z1_ref, z2_ref, z3_ref, z4_ref, o_ref):
    zsum = z1_ref[...] + z2_ref[...] + z3_ref[...] + z4_ref[...]
    o_ref[...] = (x0_ref[...] + d_ref[...] * zsum) * (1.0 / 25.0)


def _tc_base(x0, d, z1, z2, z3, z4):
    nb = NPAD // 512
    zspec = pl.BlockSpec((512, D), lambda i: (i, 0))
    dspec = pl.BlockSpec((512, 1), lambda i: (i, 0))
    return pl.pallas_call(
        _tc_base_body,
        grid=(nb,),
        in_specs=[zspec, dspec, zspec, zspec, zspec, zspec],
        out_specs=zspec,
        out_shape=jax.ShapeDtypeStruct((NPAD, D), jnp.float32),
    )(x0, d, z1, z2, z3, z4)


def _tc_proj_body(base_ref, f_ref, w_ref, o_ref):
    prod = lax.dot_general(
        f_ref[...], w_ref[...], (((1,), (1,)), ((), ())),
        preferred_element_type=jnp.float32)
    o_ref[...] = base_ref[...] + prod


def _tc_proj(base, feats, w):
    n, fdim = feats.shape
    blk = 400
    return pl.pallas_call(
        _tc_proj_body,
        grid=(n // blk,),
        in_specs=[
            pl.BlockSpec((blk, D), lambda i: (i, 0)),
            pl.BlockSpec((blk, fdim), lambda i: (i, 0)),
            pl.BlockSpec((D, fdim), lambda i: (0, 0)),
        ],
        out_specs=pl.BlockSpec((blk, D), lambda i: (i, 0)),
        out_shape=jax.ShapeDtypeStruct((n, D), jnp.float32),
    )(base, feats, w)


def kernel(edge_index, emb_users_w, emb_items_w, users_features,
           items_features, user_proj_w, item_proj_w):
    row = edge_index[0]
    col = edge_index[1]
    rowp = jnp.pad(row.reshape(NS, EDGES_T),
                   ((0, 0), (0, ETILE - EDGES_T))).reshape(-1)
    colp = jnp.pad(col.reshape(NS, EDGES_T),
                   ((0, 0), (0, ETILE - EDGES_T)),
                   constant_values=NN).reshape(-1)

    grow_f, lcol_f, degp, _hist_scratch = _prep_call(rowp, colp)
    grow2 = grow_f.reshape(NS * NCHUNK, CHUNK)
    lcol3 = lcol_f.reshape(2, NS * NCHUNK, CHUNK)
    deg0 = degp[0].reshape(NPAD, 1)
    deg1 = degp[1].reshape(NPAD, 1)

    zpad = jnp.zeros((PAD88, D), jnp.float32)
    x0 = jnp.concatenate(
        [emb_users_w[:HALF], zpad, emb_users_w[HALF:], emb_items_w, zpad],
        axis=0)

    y0, d, d2 = _tc_prep(deg0, deg1, x0)
    z1 = _layer_call(y0, grow2, lcol3)
    y1 = _tc_scale(z1, d2)
    z2 = _layer_call(y1, grow2, lcol3)
    y2 = _tc_scale(z2, d2)
    z3 = _layer_call(y2, grow2, lcol3)
    y3 = _tc_scale(z3, d2)
    z4 = _layer_call(y3, grow2, lcol3)

    base = _tc_base(x0, d, z1, z2, z3, z4)
    base_u = jnp.concatenate([base[:HALF], base[HPAD:HPAD + NU - HALF]],
                             axis=0)
    base_i = base[HPAD + NU - HALF:HPAD + NU - HALF + NI]
    out_u = _tc_proj(base_u, users_features, user_proj_w)
    out_i = _tc_proj(base_i, items_features, item_proj_w)
    return (out_u, out_i)
